# baseline (speedup 1.0000x reference)
import functools

import jax
import jax.numpy as jnp
from jax import lax
from jax.experimental import pallas as pl
from jax.experimental.pallas import tpu as pltpu
from jax.experimental.pallas import tpu_sc as plsc

_N = 10000
_E = 320000
_LAYERS = 3
_LANES = 16
_NTILES = 16
_NPAD = 10240
_EPT = _E // _NTILES
_SLICE = _NPAD // _NTILES


def _sybil_body(x_hbm, src_hbm, dst_hbm, inv_hbm, out_hbm,
                h_v, acc_v, src_v, dst_v, inv_v, tmp_v, red_v,
                stage_sp, g_sp):
    wid = lax.axis_index("s")
    base_e = wid * _EPT
    base_n = wid * _SLICE

    pltpu.sync_copy(src_hbm.at[pl.ds(base_e, _EPT)], src_v)
    pltpu.sync_copy(dst_hbm.at[pl.ds(base_e, _EPT)], dst_v)
    pltpu.sync_copy(inv_hbm.at[pl.ds(base_n, _SLICE)], inv_v)

    pltpu.sync_copy(x_hbm.at[pl.ds(base_n, _SLICE)], red_v)
    for j in range(_SLICE // _LANES):
        sl = pl.ds(j * _LANES, _LANES)
        red_v[sl] = red_v[sl] * inv_v[sl]
    pltpu.sync_copy(red_v, g_sp.at[pl.ds(base_n, _SLICE)])
    plsc.subcore_barrier()

    zero = jnp.zeros((_LANES,), jnp.float32)
    for layer in range(_LAYERS):
        pltpu.sync_copy(g_sp, h_v)

        @plsc.parallel_loop(0, _NPAD // _LANES, unroll=8)
        def clear_step(i):
            acc_v[pl.ds(i * _LANES, _LANES)] = zero

        @plsc.parallel_loop(0, _EPT // _LANES, unroll=16)
        def edge_step(i):
            sl = pl.ds(i * _LANES, _LANES)
            s = src_v[sl]
            d = dst_v[sl]
            vals = plsc.load_gather(h_v, [s])
            plsc.addupdate_scatter(acc_v, [d], vals)

        pltpu.sync_copy(acc_v, stage_sp.at[wid])
        plsc.subcore_barrier()
        pltpu.sync_copy(stage_sp.at[:, pl.ds(base_n, _SLICE)], tmp_v)
        for j in range(_SLICE // _LANES):
            sl = pl.ds(j * _LANES, _LANES)
            acc16 = tmp_v[0, sl]
            for k in range(1, _NTILES):
                acc16 = acc16 + tmp_v[k, sl]
            red_v[sl] = acc16

        if layer < _LAYERS - 1:
            for j in range(_SLICE // _LANES):
                sl = pl.ds(j * _LANES, _LANES)
                red_v[sl] = red_v[sl] * inv_v[sl]
            pltpu.sync_copy(red_v, g_sp.at[pl.ds(base_n, _SLICE)])
            plsc.subcore_barrier()
        else:
            pltpu.sync_copy(red_v, out_hbm.at[pl.ds(base_n, _SLICE)])


@jax.jit
def _sybil_call(xp, src, dst, inv):
    mesh = plsc.VectorSubcoreMesh(
        core_axis_name="c", subcore_axis_name="s", num_cores=1)
    run = functools.partial(
        pl.kernel,
        out_type=jax.ShapeDtypeStruct((_NPAD,), jnp.float32),
        mesh=mesh,
        scratch_types=[
            pltpu.VMEM((_NPAD,), jnp.float32),
            pltpu.VMEM((_NPAD,), jnp.float32),
            pltpu.VMEM((_EPT,), jnp.int32),
            pltpu.VMEM((_EPT,), jnp.int32),
            pltpu.VMEM((_SLICE,), jnp.float32),
            pltpu.VMEM((_NTILES, _SLICE), jnp.float32),
            pltpu.VMEM((_SLICE,), jnp.float32),
            pltpu.VMEM_SHARED((_NTILES, _NPAD), jnp.float32),
            pltpu.VMEM_SHARED((_NPAD,), jnp.float32),
        ],
        compiler_params=pltpu.CompilerParams(needs_layout_passes=False),
    )(_sybil_body)
    return run(xp, src, dst, inv)


def kernel(x, edge_index, neighbor_index):
    xp = jnp.zeros((_NPAD,), jnp.float32).at[:_N].set(x[:, 0])
    inv = jnp.ones((_NPAD,), jnp.float32).at[:_N].set(
        1.0 / neighbor_index.astype(jnp.float32))
    src = edge_index[0]
    dst = edge_index[1]
    out = _sybil_call(xp, src, dst, inv)
    return out[:_N, None]


# speedup vs baseline: 1.2445x; 1.2445x over previous
import functools

import jax
import jax.numpy as jnp
from jax import lax
from jax.experimental import pallas as pl
from jax.experimental.pallas import tpu as pltpu
from jax.experimental.pallas import tpu_sc as plsc

_N = 10000
_E = 320000
_LAYERS = 3
_LANES = 16
_NTILES = 16
_NPAD = 10240
_EPT = _E // _NTILES
_EALN = 128
_EBUF = 20096
_SLICE = _NPAD // _NTILES


def _sybil_body(x_hbm, edge_hbm, inv_hbm, out_hbm,
                h_v, acc_v, ed_v, inv_v, tmp_v, red_v,
                stage_sp, g_sp, sem1):
    wid = lax.axis_index("s")
    base_e = wid * _EPT
    base_n = wid * _SLICE

    start_e = pl.multiple_of(base_e // _EALN * _EALN, _EALN)
    delta = base_e - start_e
    pltpu.sync_copy(edge_hbm.at[:, pl.ds(start_e, _EBUF)], ed_v)
    pltpu.sync_copy(inv_hbm.at[pl.ds(base_n, _SLICE)], inv_v)

    pltpu.sync_copy(x_hbm.at[pl.ds(base_n, _SLICE)], red_v)
    for j in range(_SLICE // _LANES):
        sl = pl.ds(j * _LANES, _LANES)
        red_v[sl] = red_v[sl] * inv_v[sl]
    pltpu.sync_copy(red_v, g_sp.at[pl.ds(base_n, _SLICE)])
    plsc.subcore_barrier()

    zero = jnp.zeros((_LANES,), jnp.float32)
    for layer in range(_LAYERS):
        cp_h = pltpu.async_copy(g_sp, h_v, sem1)

        @plsc.parallel_loop(0, _NPAD // _LANES, unroll=8)
        def clear_step(i):
            acc_v[pl.ds(i * _LANES, _LANES)] = zero

        cp_h.wait()

        @plsc.parallel_loop(0, _EPT // _LANES, unroll=16)
        def edge_step(i):
            sl = pl.ds(delta + i * _LANES, _LANES)
            s = ed_v[0, sl]
            d = ed_v[1, sl]
            vals = plsc.load_gather(h_v, [s])
            plsc.addupdate_scatter(acc_v, [d], vals)

        pltpu.sync_copy(acc_v, stage_sp.at[wid])
        plsc.subcore_barrier()
        pltpu.sync_copy(stage_sp.at[:, pl.ds(base_n, _SLICE)], tmp_v)
        for j in range(_SLICE // _LANES):
            sl = pl.ds(j * _LANES, _LANES)
            acc16 = tmp_v[0, sl]
            for k in range(1, _NTILES):
                acc16 = acc16 + tmp_v[k, sl]
            red_v[sl] = acc16

        if layer < _LAYERS - 1:
            for j in range(_SLICE // _LANES):
                sl = pl.ds(j * _LANES, _LANES)
                red_v[sl] = red_v[sl] * inv_v[sl]
            pltpu.sync_copy(red_v, g_sp.at[pl.ds(base_n, _SLICE)])
            plsc.subcore_barrier()
        else:
            pltpu.sync_copy(red_v, out_hbm.at[pl.ds(base_n, _SLICE)])


@jax.jit
def _sybil_call(xp, edge_index, inv):
    mesh = plsc.VectorSubcoreMesh(
        core_axis_name="c", subcore_axis_name="s", num_cores=1)
    run = functools.partial(
        pl.kernel,
        out_type=jax.ShapeDtypeStruct((_NPAD,), jnp.float32),
        mesh=mesh,
        scratch_types=[
            pltpu.VMEM((_NPAD,), jnp.float32),
            pltpu.VMEM((_NPAD,), jnp.float32),
            pltpu.VMEM((2, _EBUF), jnp.int32),
            pltpu.VMEM((_SLICE,), jnp.float32),
            pltpu.VMEM((_NTILES, _SLICE), jnp.float32),
            pltpu.VMEM((_SLICE,), jnp.float32),
            pltpu.VMEM_SHARED((_NTILES, _NPAD), jnp.float32),
            pltpu.VMEM_SHARED((_NPAD,), jnp.float32),
            pltpu.SemaphoreType.DMA,
        ],
        compiler_params=pltpu.CompilerParams(needs_layout_passes=False),
    )(_sybil_body)
    return run(xp, edge_index, inv)


def kernel(x, edge_index, neighbor_index):
    xp = jnp.zeros((_NPAD,), jnp.float32).at[:_N].set(x[:, 0])
    inv = jnp.ones((_NPAD,), jnp.float32).at[:_N].set(
        1.0 / neighbor_index.astype(jnp.float32))
    out = _sybil_call(xp, edge_index, inv)
    return out[:_N, None]


# speedup vs baseline: 1.4867x; 1.1946x over previous
import functools

import jax
import jax.numpy as jnp
from jax import lax
from jax.experimental import pallas as pl
from jax.experimental.pallas import tpu as pltpu
from jax.experimental.pallas import tpu_sc as plsc

_N = 10000
_E = 320000
_LAYERS = 3
_LANES = 16
_NTILES = 16
_NPAD = 10240
_EPT = _E // _NTILES
_EALN = 128
_EBUF = 20096
_SLICE = _NPAD // _NTILES


def _sybil_body(x_hbm, edge_hbm, inv_hbm, out_hbm,
                h_v, acc_v, ed_v, inv_v, tmp_v, red_v,
                stage_sp, g_sp, sem1):
    wid = lax.axis_index("s")
    base_e = wid * _EPT
    base_n = wid * _SLICE

    start_e = pl.multiple_of(base_e // _EALN * _EALN, _EALN)
    delta = base_e - start_e
    pltpu.sync_copy(edge_hbm.at[:, pl.ds(start_e, _EBUF)], ed_v)
    pltpu.sync_copy(inv_hbm.at[pl.ds(base_n, _SLICE)], inv_v)

    pltpu.sync_copy(x_hbm.at[pl.ds(base_n, _SLICE)], red_v)

    @plsc.parallel_loop(0, _SLICE // _LANES, unroll=4)
    def scale_x_step(j):
        sl = pl.ds(j * _LANES, _LANES)
        red_v[sl] = red_v[sl] * inv_v[sl]

    pltpu.sync_copy(red_v, g_sp.at[pl.ds(base_n, _SLICE)])
    plsc.subcore_barrier()

    zero = jnp.zeros((_LANES,), jnp.float32)
    for layer in range(_LAYERS):
        cp_h = pltpu.async_copy(g_sp, h_v, sem1)

        @plsc.parallel_loop(0, _NPAD // _LANES, unroll=8)
        def clear_step(i):
            acc_v[pl.ds(i * _LANES, _LANES)] = zero

        cp_h.wait()

        @plsc.parallel_loop(0, _EPT // _LANES, unroll=16)
        def edge_step(i):
            sl = pl.ds(delta + i * _LANES, _LANES)
            s = ed_v[0, sl]
            d = ed_v[1, sl]
            vals = plsc.load_gather(h_v, [s])
            plsc.addupdate_scatter(acc_v, [d], vals)

        pltpu.sync_copy(acc_v, stage_sp.at[wid])
        plsc.subcore_barrier()
        pltpu.sync_copy(stage_sp.at[:, pl.ds(base_n, _SLICE)], tmp_v)

        if layer < _LAYERS - 1:
            @plsc.parallel_loop(0, _SLICE // _LANES, unroll=2)
            def red_scale_step(j):
                sl = pl.ds(j * _LANES, _LANES)
                acc16 = tmp_v[0, sl]
                for k in range(1, _NTILES):
                    acc16 = acc16 + tmp_v[k, sl]
                red_v[sl] = acc16 * inv_v[sl]

            pltpu.sync_copy(red_v, g_sp.at[pl.ds(base_n, _SLICE)])
            plsc.subcore_barrier()
        else:
            @plsc.parallel_loop(0, _SLICE // _LANES, unroll=2)
            def red_step(j):
                sl = pl.ds(j * _LANES, _LANES)
                acc16 = tmp_v[0, sl]
                for k in range(1, _NTILES):
                    acc16 = acc16 + tmp_v[k, sl]
                red_v[sl] = acc16

            pltpu.sync_copy(red_v, out_hbm.at[pl.ds(base_n, _SLICE)])


@jax.jit
def _sybil_call(xp, edge_index, inv):
    mesh = plsc.VectorSubcoreMesh(
        core_axis_name="c", subcore_axis_name="s", num_cores=1)
    run = functools.partial(
        pl.kernel,
        out_type=jax.ShapeDtypeStruct((_NPAD,), jnp.float32),
        mesh=mesh,
        scratch_types=[
            pltpu.VMEM((_NPAD,), jnp.float32),
            pltpu.VMEM((_NPAD,), jnp.float32),
            pltpu.VMEM((2, _EBUF), jnp.int32),
            pltpu.VMEM((_SLICE,), jnp.float32),
            pltpu.VMEM((_NTILES, _SLICE), jnp.float32),
            pltpu.VMEM((_SLICE,), jnp.float32),
            pltpu.VMEM_SHARED((_NTILES, _NPAD), jnp.float32),
            pltpu.VMEM_SHARED((_NPAD,), jnp.float32),
            pltpu.SemaphoreType.DMA,
        ],
        compiler_params=pltpu.CompilerParams(needs_layout_passes=False),
    )(_sybil_body)
    return run(xp, edge_index, inv)


def kernel(x, edge_index, neighbor_index):
    xp = jnp.zeros((_NPAD,), jnp.float32).at[:_N].set(x[:, 0])
    inv = jnp.ones((_NPAD,), jnp.float32).at[:_N].set(
        1.0 / neighbor_index.astype(jnp.float32))
    out = _sybil_call(xp, edge_index, inv)
    return out[:_N, None]
